# baseline (speedup 1.0000x reference)
import functools

import jax
import jax.numpy as jnp
from jax import lax
from jax.experimental import pallas as pl
from jax.experimental.pallas import tpu as pltpu
from jax.experimental.pallas import tpu_sc as plsc

B, P, H, W = 8, 100000, 512, 512
HW = H * W
BP = B * P
PMAIN = 99968
PREM = P - PMAIN
PADP = 100096
BPP = B * PADP
ROWS_P = BPP // 128
NC, NS = 2, 16
SLI = HW // NS
CNT = PADP // NS

_mesh = plsc.VectorSubcoreMesh(core_axis_name="c", subcore_axis_name="s")


def _idx_body(xam, xar, yam, yar, xbm, xbr, ybm, ybr, om, orr,
              ia_ref, ib_ref, op_ref):
    lane = lax.broadcasted_iota(jnp.int32, (1, 128), 1).reshape(128)
    valid = lane < PREM

    def side(xm, ym, xr, yr):
        main = ym[...] * W + xm[...]
        remv = yr[...] * W + xr[...]
        pieces = []
        for b in range(B):
            base = (b % 2) * HW
            pieces.append((main[b : b + 1] + base).reshape(PMAIN))
            pieces.append(jnp.where(
                valid, remv[b : b + 1].reshape(128) + base, 0))
        return jnp.concatenate(pieces, axis=0)

    ia_ref[...] = side(xam, yam, xar, yar)
    ib_ref[...] = side(xbm, ybm, xbr, ybr)
    opieces = []
    for b in range(B):
        opieces.append(om[b : b + 1].reshape(PMAIN))
        opieces.append(jnp.where(valid, orr[b : b + 1].reshape(128), 0.0))
    op_ref[...] = jnp.concatenate(opieces, axis=0)


def _in_pair():
    return (
        pl.BlockSpec((B, PMAIN), lambda i: (0, 0)),
        pl.BlockSpec((B, 128), lambda i: (0, PMAIN // 128)),
    )


_idx_call = pl.pallas_call(
    _idx_body,
    grid=(1,),
    out_specs=(
        pl.BlockSpec((BPP,), lambda i: (0,)),
        pl.BlockSpec((BPP,), lambda i: (0,)),
        pl.BlockSpec((BPP,), lambda i: (0,)),
    ),
    in_specs=[
        *_in_pair(),
        *_in_pair(),
        *_in_pair(),
        *_in_pair(),
        *_in_pair(),
    ],
    out_shape=(
        jax.ShapeDtypeStruct((BPP,), jnp.int32),
        jax.ShapeDtypeStruct((BPP,), jnp.int32),
        jax.ShapeDtypeStruct((BPP,), jnp.float32),
    ),
)


@functools.partial(
    pl.kernel,
    mesh=_mesh,
    out_type=(
        jax.ShapeDtypeStruct((BPP,), jnp.float32),
        jax.ShapeDtypeStruct((BPP,), jnp.float32),
    ),
    scratch_types=[
        pltpu.VMEM((2 * CNT,), jnp.int32),
        pltpu.VMEM((2 * CNT,), jnp.int32),
        pltpu.VMEM((2 * CNT,), jnp.float32),
        pltpu.VMEM((2 * CNT,), jnp.float32),
        pltpu.VMEM_SHARED((2 * HW,), jnp.float32),
        pltpu.SemaphoreType.DMA,
        pltpu.SemaphoreType.DMA,
        pltpu.SemaphoreType.DMA,
    ],
)
def _sc_gather(depth, idx_a, idx_b, out_a, out_b, via, vib, z_a, z_b,
               spmem, sem_s, sem_a, sem_b):
    c = lax.axis_index("c")
    s = lax.axis_index("s")
    C2 = 2 * CNT
    SL2 = 2 * HW // NS

    def stage(k):
        dbase = (4 * c + 2 * k) * HW + s * SL2
        return pltpu.async_copy(
            depth.at[pl.ds(pl.multiple_of(dbase, 8), SL2)],
            spmem.at[pl.ds(s * SL2, SL2)], sem_s)

    def gathers(k):
        gb = pl.multiple_of((4 * c + 2 * k) * PADP + s * C2, 8)
        pltpu.sync_copy(idx_a.at[pl.ds(gb, C2)], via)
        pltpu.sync_copy(idx_b.at[pl.ds(gb, C2)], vib)
        cp_a = pltpu.async_copy(spmem.at[via], z_a, sem_a)
        cp_b = pltpu.async_copy(spmem.at[vib], z_b, sem_b)
        cp_a.wait()
        pltpu.sync_copy(z_a, out_a.at[pl.ds(gb, C2)])
        cp_b.wait()
        pltpu.sync_copy(z_b, out_b.at[pl.ds(gb, C2)])

    stage(0).wait()
    plsc.subcore_barrier()
    gathers(0)
    plsc.subcore_barrier()
    stage(1).wait()
    plsc.subcore_barrier()
    gathers(1)


_LGRID = 2
_LROWS = ROWS_P // _LGRID


def _loss_body(za_ref, zb_ref, g_ref, o_ref):
    d = za_ref[...] - zb_ref[...]
    g = g_ref[...]
    mask = jnp.abs(g)
    loss = mask * jnp.log(1.0 + jnp.exp(-g * d)) + (1.0 - mask) * (d * d)
    o_ref[0, 0] = jnp.sum(loss) / BP


_loss_call = pl.pallas_call(
    _loss_body,
    out_shape=jax.ShapeDtypeStruct((1, 1), jnp.float32),
    out_specs=pl.BlockSpec(memory_space=pltpu.SMEM),
)


def kernel(input, x_A, y_A, x_B, y_B, ordinal_relation):
    depth = input.reshape(B * H * W)
    ia, ib, ordp = _idx_call(x_A, x_A, y_A, y_A, x_B, x_B, y_B, y_B,
                             ordinal_relation, ordinal_relation)
    z_a, z_b = _sc_gather(depth, ia, ib)
    out = _loss_call(
        z_a.reshape(ROWS_P, 128),
        z_b.reshape(ROWS_P, 128),
        ordp.reshape(ROWS_P, 128),
    )
    return out.reshape(1)


# speedup vs baseline: 1.0631x; 1.0631x over previous
import functools

import jax
import jax.numpy as jnp
from jax import lax
from jax.experimental import pallas as pl
from jax.experimental.pallas import tpu as pltpu
from jax.experimental.pallas import tpu_sc as plsc

B, P, H, W = 8, 100000, 512, 512
HW = H * W
BP = B * P
PMAIN = 99968
PREM = P - PMAIN
PADP = 100096
BPP = B * PADP
ROWS_P = BPP // 128
NC, NS = 2, 16
SLI = HW // NS
CNT = PADP // NS

_mesh = plsc.VectorSubcoreMesh(core_axis_name="c", subcore_axis_name="s")


def _idx_body(xam, xar, yam, yar, xbm, xbr, ybm, ybr, om, orr,
              ia_ref, ib_ref, op_ref):
    lane = lax.broadcasted_iota(jnp.int32, (1, 128), 1).reshape(128)
    valid = lane < PREM

    def side(xm, ym, xr, yr):
        main = ym[...] * W + xm[...]
        remv = yr[...] * W + xr[...]
        pieces = []
        for b in range(B):
            base = (b % 2) * HW
            pieces.append((main[b : b + 1] + base).reshape(PMAIN))
            pieces.append(jnp.where(
                valid, remv[b : b + 1].reshape(128) + base, 0))
        return jnp.concatenate(pieces, axis=0)

    ia_ref[...] = side(xam, yam, xar, yar)
    ib_ref[...] = side(xbm, ybm, xbr, ybr)
    opieces = []
    for b in range(B):
        opieces.append(om[b : b + 1].reshape(PMAIN))
        opieces.append(jnp.where(valid, orr[b : b + 1].reshape(128), 0.0))
    op_ref[...] = jnp.concatenate(opieces, axis=0)


def _in_pair():
    return (
        pl.BlockSpec((B, PMAIN), lambda i: (0, 0)),
        pl.BlockSpec((B, 128), lambda i: (0, PMAIN // 128)),
    )


_idx_call = pl.pallas_call(
    _idx_body,
    grid=(1,),
    out_specs=(
        pl.BlockSpec((BPP,), lambda i: (0,)),
        pl.BlockSpec((BPP,), lambda i: (0,)),
        pl.BlockSpec((BPP,), lambda i: (0,)),
    ),
    in_specs=[
        *_in_pair(),
        *_in_pair(),
        *_in_pair(),
        *_in_pair(),
        *_in_pair(),
    ],
    out_shape=(
        jax.ShapeDtypeStruct((BPP,), jnp.int32),
        jax.ShapeDtypeStruct((BPP,), jnp.int32),
        jax.ShapeDtypeStruct((BPP,), jnp.float32),
    ),
)


@functools.partial(
    pl.kernel,
    mesh=_mesh,
    out_type=(
        jax.ShapeDtypeStruct((BPP,), jnp.float32),
        jax.ShapeDtypeStruct((BPP,), jnp.float32),
    ),
    scratch_types=[
        pltpu.VMEM((2 * CNT,), jnp.int32),
        pltpu.VMEM((2 * CNT,), jnp.int32),
        pltpu.VMEM((2 * CNT,), jnp.int32),
        pltpu.VMEM((2 * CNT,), jnp.int32),
        pltpu.VMEM((2 * CNT,), jnp.float32),
        pltpu.VMEM((2 * CNT,), jnp.float32),
        pltpu.VMEM_SHARED((2 * HW,), jnp.float32),
        pltpu.SemaphoreType.DMA,
        pltpu.SemaphoreType.DMA,
        pltpu.SemaphoreType.DMA,
    ],
)
def _sc_gather(depth, idx_a, idx_b, out_a, out_b, via0, vib0, via1, vib1,
               z_a, z_b, spmem, sem_s, sem_a, sem_b):
    c = lax.axis_index("c")
    s = lax.axis_index("s")
    C2 = 2 * CNT
    SL2 = 2 * HW // NS
    bufs = ((via0, vib0), (via1, vib1))

    def stage(k):
        dbase = (4 * c + 2 * k) * HW + s * SL2
        return pltpu.async_copy(
            depth.at[pl.ds(pl.multiple_of(dbase, 8), SL2)],
            spmem.at[pl.ds(s * SL2, SL2)], sem_s)

    def ldidx(k):
        gb = pl.multiple_of((4 * c + 2 * k) * PADP + s * C2, 8)
        va, vb = bufs[k]
        pltpu.sync_copy(idx_a.at[pl.ds(gb, C2)], va)
        pltpu.sync_copy(idx_b.at[pl.ds(gb, C2)], vb)

    def gathers(k):
        gb = pl.multiple_of((4 * c + 2 * k) * PADP + s * C2, 8)
        va, vb = bufs[k]
        cp_a = pltpu.async_copy(spmem.at[va], z_a, sem_a)
        cp_b = pltpu.async_copy(spmem.at[vb], z_b, sem_b)
        if k == 0:
            ldidx(1)
        cp_a.wait()
        pltpu.sync_copy(z_a, out_a.at[pl.ds(gb, C2)])
        cp_b.wait()
        pltpu.sync_copy(z_b, out_b.at[pl.ds(gb, C2)])

    st = stage(0)
    ldidx(0)
    st.wait()
    plsc.subcore_barrier()
    gathers(0)
    plsc.subcore_barrier()
    stage(1).wait()
    plsc.subcore_barrier()
    gathers(1)


_LGRID = 2
_LROWS = ROWS_P // _LGRID


def _loss_body(za_ref, zb_ref, g_ref, o_ref):
    d = za_ref[...] - zb_ref[...]
    g = g_ref[...]
    mask = jnp.abs(g)
    loss = mask * jnp.log(1.0 + jnp.exp(-g * d)) + (1.0 - mask) * (d * d)
    o_ref[0, 0] = jnp.sum(loss) / BP


_loss_call = pl.pallas_call(
    _loss_body,
    out_shape=jax.ShapeDtypeStruct((1, 1), jnp.float32),
    out_specs=pl.BlockSpec(memory_space=pltpu.SMEM),
)


def kernel(input, x_A, y_A, x_B, y_B, ordinal_relation):
    depth = input.reshape(B * H * W)
    ia, ib, ordp = _idx_call(x_A, x_A, y_A, y_A, x_B, x_B, y_B, y_B,
                             ordinal_relation, ordinal_relation)
    z_a, z_b = _sc_gather(depth, ia, ib)
    out = _loss_call(
        z_a.reshape(ROWS_P, 128),
        z_b.reshape(ROWS_P, 128),
        ordp.reshape(ROWS_P, 128),
    )
    return out.reshape(1)
